# baseline (speedup 1.0000x reference)
import functools

import jax
import jax.numpy as jnp
from jax import lax
from jax.experimental import pallas as pl
from jax.experimental.pallas import tpu as pltpu, tpu_sc as plsc

G = 1000
E = 64
M = 4
S = 8
B = 1024
T = 20

DCAT = 2 * E
DOUT = DCAT + 16
NW = 32
RPT = B // NW
IPT = RPT * T
NCH = IPT // 128


def _sc_gather_sums(xf, table, vocab_b):
    mesh = plsc.VectorSubcoreMesh(
        core_axis_name="c", subcore_axis_name="s", num_cores=2, num_subcores=16)

    @functools.partial(
        pl.kernel,
        out_type=jax.ShapeDtypeStruct((B, DOUT), jnp.float32),
        mesh=mesh,
        compiler_params=pltpu.CompilerParams(needs_layout_passes=False),
        scratch_types=[
            pltpu.VMEM((IPT,), jnp.int32),
            pltpu.VMEM((NCH, 128), jnp.int32),
            pltpu.VMEM((IPT, DCAT), jnp.float32),
            pltpu.VMEM((RPT, DOUT), jnp.float32),
            pltpu.VMEM((G,), jnp.float32),
            pltpu.VMEM_SHARED((G, DCAT), jnp.float32),
            pltpu.SemaphoreType.DMA,
        ],
    )
    def k(x_hbm, table_hbm, vb_hbm, sums_hbm,
          idxf_v, idx_v, rows_v, acc_v, vb_v, tab_sh, sem):
        nc = 2
        wid = lax.axis_index("s") * nc + lax.axis_index("c")
        sid = lax.axis_index("s")
        for i in range(16):
            rows = 64 if i < 15 else G - 15 * 64

            @pl.when(sid == i)
            def _(i=i, rows=rows):
                pltpu.sync_copy(table_hbm.at[pl.ds(i * 64, rows)],
                                tab_sh.at[pl.ds(i * 64, rows)])

        pltpu.sync_copy(x_hbm.at[wid], idxf_v)
        pltpu.sync_copy(vb_hbm, vb_v)
        for kk in range(IPT // 16):
            idx_v[kk // 8, pl.ds((kk % 8) * 16, 16)] = idxf_v[pl.ds(kk * 16, 16)]
        plsc.subcore_barrier()

        copies = []
        for j in range(NCH):
            copies.append(
                pltpu.async_copy(
                    tab_sh.at[idx_v.at[j]],
                    rows_v.at[pl.ds(j * 128, 128)],
                    sem,
                )
            )

        lane = lax.broadcasted_iota(jnp.int32, (16,), 0)

        @plsc.parallel_loop(0, RPT, unroll=4)
        def _(r):
            v1 = plsc.load_gather(vb_v, [idxf_v[pl.ds(r * T, 16)]])
            v2 = plsc.load_gather(vb_v, [idxf_v[pl.ds(r * T + 4, 16)]])
            v2 = jnp.where(lane >= 12, v2, 0.0)
            s = jax.lax.reduce_sum_p.bind(v1 + v2, axes=(0,))
            acc_v[r, pl.ds(DCAT, 16)] = jnp.where(lane == 0, s, 0.0)

        for c in copies:
            c.wait()

        @plsc.parallel_loop(0, RPT, unroll=4)
        def _(r):
            base = r * T
            for c in range(DCAT // 16):
                sl = pl.ds(c * 16, 16)
                acc = rows_v[base, sl]
                for t in range(1, T):
                    acc = acc + rows_v[base + t, sl]
                acc_v[r, sl] = acc

        pltpu.sync_copy(acc_v, sums_hbm.at[pl.ds(wid * RPT, RPT)])

    return k(xf, table, vocab_b)


def _tc_body(sums_ref, w_ref, b_ref, init_ref, tr_ref, out_ref):
    inv_t = 1.0 / float(T)
    sums = sums_ref[:]
    me = sums[:, 0:E] * inv_t
    gw = sums[:, E:2 * E]
    gb = sums[:, DCAT:DCAT + 1]

    logits = lax.dot_general(
        me, w_ref[:], (((1,), (1,)), ((), ())),
        preferred_element_type=jnp.float32,
    ) + b_ref[:]
    mx = jnp.max(logits, axis=1, keepdims=True)
    lse = mx + jnp.log(jnp.sum(jnp.exp(logits - mx), axis=1, keepdims=True))
    edot = jnp.sum(me * gw, axis=1, keepdims=True) + gb

    row = lax.broadcasted_iota(jnp.int32, (M * S, S), 0)
    lane = lax.broadcasted_iota(jnp.int32, (M * S, S), 1)
    mask8 = (row % S) == lane
    ssel = jnp.where(
        (lax.broadcasted_iota(jnp.int32, (M * S, M * S), 0) // S)
        == (lax.broadcasted_iota(jnp.int32, (M * S, M * S), 1) // S),
        1.0, 0.0).astype(jnp.float32)

    def _lse_rows(z):
        m = jnp.max(z, axis=1, keepdims=True)
        return m + jnp.log(jnp.sum(jnp.exp(z - m), axis=1, keepdims=True))

    lt = tr_ref[:] * 100.0
    lt = lt - _lse_rows(lt)
    ab = init_ref[:] * 100.0
    ab = ab - _lse_rows(ab)
    an = ab[:, 0:1]
    for i in range(T):
        an = _lse_rows(lt + ab)
        if i < T - 1:
            at = jnp.where(mask8, jnp.broadcast_to(an, (M * S, S)), 0.0)
            ab = lax.dot_general(
                ssel, at, (((1,), (0,)), ((), ())),
                preferred_element_type=jnp.float32,
            )
    ad = an * inv_t
    cmx = jnp.max(ad, axis=0, keepdims=True)
    cc = cmx + jnp.log(jnp.sum(jnp.exp(ad - cmx), axis=0, keepdims=True))

    out_ref[:] = edot * inv_t - lse + cc


def kernel(zi, x, embed_table, vocab_W, vocab_b, init_dist, transition, state_vect):
    del zi, state_vect

    xf = x.reshape(NW, IPT).astype(jnp.int32)
    table = jnp.concatenate([embed_table, vocab_W], axis=1)
    sums = _sc_gather_sums(xf, table, vocab_b)

    init_rep = jnp.broadcast_to(
        init_dist.reshape(M, 1, S), (M, S, S)).reshape(M * S, S)
    tr_rep = jnp.transpose(
        transition.reshape(M, S, S), (0, 2, 1)).reshape(M * S, S)

    out = pl.pallas_call(
        _tc_body,
        out_shape=jax.ShapeDtypeStruct((B, 1), jnp.float32),
    )(sums, vocab_W, vocab_b.reshape(1, G), init_rep, tr_rep)
    return out


# speedup vs baseline: 1.0429x; 1.0429x over previous
import functools

import jax
import jax.numpy as jnp
from jax import lax
from jax.experimental import pallas as pl
from jax.experimental.pallas import tpu as pltpu, tpu_sc as plsc

G = 1000
E = 64
M = 4
S = 8
B = 1024
T = 20

DCAT = 2 * E
DOUT = DCAT + 16
NW = 32
RPT = B // NW
IPT = RPT * T
NCH = IPT // 128


def _sc_gather_sums(xf, table, vocab_b):
    mesh = plsc.VectorSubcoreMesh(
        core_axis_name="c", subcore_axis_name="s", num_cores=2, num_subcores=16)

    @functools.partial(
        pl.kernel,
        out_type=jax.ShapeDtypeStruct((B, DOUT), jnp.float32),
        mesh=mesh,
        compiler_params=pltpu.CompilerParams(needs_layout_passes=False),
        scratch_types=[
            pltpu.VMEM((IPT,), jnp.int32),
            pltpu.VMEM((NCH, 128), jnp.int32),
            pltpu.VMEM((IPT, DCAT), jnp.float32),
            pltpu.VMEM((RPT, DOUT), jnp.float32),
            pltpu.VMEM((G,), jnp.float32),
            pltpu.VMEM_SHARED((G, DCAT), jnp.float32),
            pltpu.SemaphoreType.DMA,
        ],
    )
    def k(x_hbm, table_hbm, vb_hbm, sums_hbm,
          idxf_v, idx_v, rows_v, acc_v, vb_v, tab_sh, sem):
        nc = 2
        wid = lax.axis_index("s") * nc + lax.axis_index("c")
        sid = lax.axis_index("s")
        for i in range(16):
            rows = 64 if i < 15 else G - 15 * 64

            @pl.when(sid == i)
            def _(i=i, rows=rows):
                pltpu.sync_copy(table_hbm.at[pl.ds(i * 64, rows)],
                                tab_sh.at[pl.ds(i * 64, rows)])

        pltpu.sync_copy(x_hbm.at[wid], idxf_v)
        pltpu.sync_copy(vb_hbm, vb_v)
        for kk in range(IPT // 16):
            idx_v[kk // 8, pl.ds((kk % 8) * 16, 16)] = idxf_v[pl.ds(kk * 16, 16)]
        plsc.subcore_barrier()

        copies = []
        for j in range(NCH):
            copies.append(
                pltpu.async_copy(
                    tab_sh.at[idx_v.at[j]],
                    rows_v.at[pl.ds(j * 128, 128)],
                    sem,
                )
            )

        lane = lax.broadcasted_iota(jnp.int32, (16,), 0)

        @plsc.parallel_loop(0, RPT, unroll=2)
        def _(r):
            v1 = plsc.load_gather(vb_v, [idxf_v[pl.ds(r * T, 16)]])
            v2 = plsc.load_gather(vb_v, [idxf_v[pl.ds(r * T + 4, 16)]])
            v2 = jnp.where(lane >= 12, v2, 0.0)
            s = jax.lax.reduce_sum_p.bind(v1 + v2, axes=(0,))
            acc_v[r, pl.ds(DCAT, 16)] = jnp.where(lane == 0, s, 0.0)

        for c in copies:
            c.wait()

        @plsc.parallel_loop(0, RPT, unroll=2)
        def _(r):
            base = r * T
            for c in range(DCAT // 16):
                sl = pl.ds(c * 16, 16)
                acc = rows_v[base, sl]
                for t in range(1, T):
                    acc = acc + rows_v[base + t, sl]
                acc_v[r, sl] = acc

        pltpu.sync_copy(acc_v, sums_hbm.at[pl.ds(wid * RPT, RPT)])

    return k(xf, table, vocab_b)


def _tc_body(sums_ref, w_ref, b_ref, init_ref, tr_ref, out_ref):
    inv_t = 1.0 / float(T)
    sums = sums_ref[:]
    me = sums[:, 0:E] * inv_t
    gw = sums[:, E:2 * E]
    gb = sums[:, DCAT:DCAT + 1]

    logits = lax.dot_general(
        me, w_ref[:], (((1,), (1,)), ((), ())),
        preferred_element_type=jnp.float32,
    ) + b_ref[:]
    mx = jnp.max(logits, axis=1, keepdims=True)
    lse = mx + jnp.log(jnp.sum(jnp.exp(logits - mx), axis=1, keepdims=True))
    edot = jnp.sum(me * gw, axis=1, keepdims=True) + gb

    row = lax.broadcasted_iota(jnp.int32, (M * S, S), 0)
    lane = lax.broadcasted_iota(jnp.int32, (M * S, S), 1)
    mask8 = (row % S) == lane
    ssel = jnp.where(
        (lax.broadcasted_iota(jnp.int32, (M * S, M * S), 0) // S)
        == (lax.broadcasted_iota(jnp.int32, (M * S, M * S), 1) // S),
        1.0, 0.0).astype(jnp.float32)

    def _lse_rows(z):
        m = jnp.max(z, axis=1, keepdims=True)
        return m + jnp.log(jnp.sum(jnp.exp(z - m), axis=1, keepdims=True))

    lt = tr_ref[:] * 100.0
    lt = lt - _lse_rows(lt)
    ab = init_ref[:] * 100.0
    ab = ab - _lse_rows(ab)
    an = ab[:, 0:1]
    for i in range(T):
        an = _lse_rows(lt + ab)
        if i < T - 1:
            at = jnp.where(mask8, jnp.broadcast_to(an, (M * S, S)), 0.0)
            ab = lax.dot_general(
                ssel, at, (((1,), (0,)), ((), ())),
                preferred_element_type=jnp.float32,
            )
    ad = an * inv_t
    cmx = jnp.max(ad, axis=0, keepdims=True)
    cc = cmx + jnp.log(jnp.sum(jnp.exp(ad - cmx), axis=0, keepdims=True))

    out_ref[:] = edot * inv_t - lse + cc


def kernel(zi, x, embed_table, vocab_W, vocab_b, init_dist, transition, state_vect):
    del zi, state_vect

    xf = x.reshape(NW, IPT).astype(jnp.int32)
    table = jnp.concatenate([embed_table, vocab_W], axis=1)
    sums = _sc_gather_sums(xf, table, vocab_b)

    init_rep = jnp.broadcast_to(
        init_dist.reshape(M, 1, S), (M, S, S)).reshape(M * S, S)
    tr_rep = jnp.transpose(
        transition.reshape(M, S, S), (0, 2, 1)).reshape(M * S, S)

    out = pl.pallas_call(
        _tc_body,
        out_shape=jax.ShapeDtypeStruct((B, 1), jnp.float32),
    )(sums, vocab_W, vocab_b.reshape(1, G), init_rep, tr_rep)
    return out


# speedup vs baseline: 1.1626x; 1.1148x over previous
import functools

import jax
import jax.numpy as jnp
from jax import lax
from jax.experimental import pallas as pl
from jax.experimental.pallas import tpu as pltpu, tpu_sc as plsc

G = 1000
E = 64
M = 4
S = 8
B = 1024
T = 20

DCAT = 2 * E
DOUT = DCAT + 16
NW = 32
RPT = B // NW
IPT = RPT * T
NCH = IPT // 128


def _sc_gather_sums(xf, table, vocab_b):
    mesh = plsc.VectorSubcoreMesh(
        core_axis_name="c", subcore_axis_name="s", num_cores=2, num_subcores=16)

    @functools.partial(
        pl.kernel,
        out_type=jax.ShapeDtypeStruct((B, DOUT), jnp.float32),
        mesh=mesh,
        compiler_params=pltpu.CompilerParams(needs_layout_passes=False),
        scratch_types=[
            pltpu.VMEM((IPT,), jnp.int32),
            pltpu.VMEM((NCH, 128), jnp.int32),
            pltpu.VMEM((IPT, DCAT), jnp.float32),
            pltpu.VMEM((RPT, DOUT), jnp.float32),
            pltpu.VMEM((G,), jnp.float32),
            pltpu.VMEM_SHARED((G, DCAT), jnp.float32),
            pltpu.SemaphoreType.DMA,
        ],
    )
    def k(x_hbm, table_hbm, vb_hbm, sums_hbm,
          idxf_v, idx_v, rows_v, acc_v, vb_v, tab_sh, sem):
        nc = 2
        wid = lax.axis_index("s") * nc + lax.axis_index("c")
        sid = lax.axis_index("s")
        for i in range(16):
            rows = 64 if i < 15 else G - 15 * 64

            @pl.when(sid == i)
            def _(i=i, rows=rows):
                pltpu.sync_copy(table_hbm.at[pl.ds(i * 64, rows)],
                                tab_sh.at[pl.ds(i * 64, rows)])

        pltpu.sync_copy(x_hbm.at[wid], idxf_v)
        pltpu.sync_copy(vb_hbm, vb_v)
        for kk in range(IPT // 16):
            idx_v[kk // 8, pl.ds((kk % 8) * 16, 16)] = idxf_v[pl.ds(kk * 16, 16)]
        plsc.subcore_barrier()

        copies = []
        for j in range(NCH):
            copies.append(
                pltpu.async_copy(
                    tab_sh.at[idx_v.at[j]],
                    rows_v.at[pl.ds(j * 128, 128)],
                    sem,
                )
            )

        lane = lax.broadcasted_iota(jnp.int32, (16,), 0)

        @plsc.parallel_loop(0, RPT, unroll=2)
        def _(r):
            v1 = plsc.load_gather(vb_v, [idxf_v[pl.ds(r * T, 16)]])
            v2 = plsc.load_gather(vb_v, [idxf_v[pl.ds(r * T + 4, 16)]])
            v2 = jnp.where(lane >= 12, v2, 0.0)
            s = jax.lax.reduce_sum_p.bind(v1 + v2, axes=(0,))
            acc_v[r, pl.ds(DCAT, 16)] = jnp.where(lane == 0, s, 0.0)

        for c in copies:
            c.wait()

        @plsc.parallel_loop(0, RPT, unroll=2)
        def _(r):
            base = r * T
            for c in range(DCAT // 16):
                sl = pl.ds(c * 16, 16)
                acc = rows_v[base, sl]
                for t in range(1, T):
                    acc = acc + rows_v[base + t, sl]
                acc_v[r, sl] = acc

        pltpu.sync_copy(acc_v, sums_hbm.at[pl.ds(wid * RPT, RPT)])

    return k(xf, table, vocab_b)


def _tc_body(sums_ref, w_ref, b_ref, cc_ref, out_ref):
    inv_t = 1.0 / float(T)
    sums = sums_ref[:]
    me = sums[:, 0:E] * inv_t
    gw = sums[:, E:2 * E]
    gb = sums[:, DCAT:DCAT + 1]

    logits = lax.dot_general(
        me, w_ref[:], (((1,), (1,)), ((), ())),
        preferred_element_type=jnp.float32,
    ) + b_ref[:]
    mx = jnp.max(logits, axis=1, keepdims=True)
    lse = mx + jnp.log(jnp.sum(jnp.exp(logits - mx), axis=1, keepdims=True))
    edot = jnp.sum(me * gw, axis=1, keepdims=True) + gb

    out_ref[:] = edot * inv_t - lse + cc_ref[0, 0]


def _c_body(init_ref, tr_ref, cc_ref):
    inv_t = 1.0 / float(T)
    row = lax.broadcasted_iota(jnp.int32, (M * S, S), 0)
    lane = lax.broadcasted_iota(jnp.int32, (M * S, S), 1)
    mask8 = (row % S) == lane
    ssel = jnp.where(
        (lax.broadcasted_iota(jnp.int32, (M * S, M * S), 0) // S)
        == (lax.broadcasted_iota(jnp.int32, (M * S, M * S), 1) // S),
        1.0, 0.0).astype(jnp.float32)

    def _lse_rows(z):
        m = jnp.max(z, axis=1, keepdims=True)
        return m + jnp.log(jnp.sum(jnp.exp(z - m), axis=1, keepdims=True))

    lt = tr_ref[:] * 100.0
    lt = lt - _lse_rows(lt)
    ab = init_ref[:] * 100.0
    ab = ab - _lse_rows(ab)
    an = ab[:, 0:1]
    for i in range(T):
        an = _lse_rows(lt + ab)
        if i < T - 1:
            at = jnp.where(mask8, jnp.broadcast_to(an, (M * S, S)), 0.0)
            ab = lax.dot_general(
                ssel, at, (((1,), (0,)), ((), ())),
                preferred_element_type=jnp.float32,
            )
    ad = an * inv_t
    cmx = jnp.max(ad, axis=0, keepdims=True)
    cc_ref[:] = cmx + jnp.log(
        jnp.sum(jnp.exp(ad - cmx), axis=0, keepdims=True))


def kernel(zi, x, embed_table, vocab_W, vocab_b, init_dist, transition, state_vect):
    del zi, state_vect

    xf = x.reshape(NW, IPT).astype(jnp.int32)
    table = jnp.concatenate([embed_table, vocab_W], axis=1)
    sums = _sc_gather_sums(xf, table, vocab_b)

    init_rep = jnp.broadcast_to(
        init_dist.reshape(M, 1, S), (M, S, S)).reshape(M * S, S)
    tr_rep = jnp.transpose(
        transition.reshape(M, S, S), (0, 2, 1)).reshape(M * S, S)

    cc = pl.pallas_call(
        _c_body,
        out_shape=jax.ShapeDtypeStruct((1, 1), jnp.float32),
    )(init_rep, tr_rep)

    out = pl.pallas_call(
        _tc_body,
        out_shape=jax.ShapeDtypeStruct((B, 1), jnp.float32),
    )(sums, vocab_W, vocab_b.reshape(1, G), cc)
    return out
